# baseline (speedup 1.0000x reference)
import functools

import jax
import jax.numpy as jnp
from jax import lax
from jax.experimental import pallas as pl
from jax.experimental.pallas import tpu as pltpu
from jax.experimental.pallas import tpu_sc as plsc

VOCAB = 1000000
HIDDEN = 64
LABELS = 50
BATCH = 4096
SEQ = 200

NC = 2
NS = 16
NW = NC * NS
ROWS_PER_W = BATCH // NW
CHUNK = 100
CHUNKS_PER_ROW = SEQ // CHUNK
CHUNKS_PER_W = ROWS_PER_W * CHUNKS_PER_ROW
NVEC = HIDDEN // 16


def _pool_body(idx_hbm, emb_hbm, h_hbm, idx_v, rows_v, h_v, sem):
    wid = lax.axis_index("s") * NC + lax.axis_index("c")

    pltpu.sync_copy(idx_hbm.at[pl.ds(wid * CHUNKS_PER_W, CHUNKS_PER_W)], idx_v)

    inv = jnp.float32(1.0 / SEQ)

    def row_body(r, carry):
        c0 = r * CHUNKS_PER_ROW
        d0 = pltpu.async_copy(
            emb_hbm.at[idx_v.at[c0]], rows_v.at[pl.ds(0, CHUNK)], sem)
        d1 = pltpu.async_copy(
            emb_hbm.at[idx_v.at[c0 + 1]], rows_v.at[pl.ds(CHUNK, CHUNK)], sem)
        d0.wait()
        d1.wait()

        def acc_body(j, acc):
            return tuple(
                acc[d] + rows_v[j, pl.ds(16 * d, 16)] for d in range(NVEC))

        acc = lax.fori_loop(
            0, SEQ, acc_body,
            tuple(jnp.zeros((16,), jnp.float32) for _ in range(NVEC)),
            unroll=8)
        for d in range(NVEC):
            h_v[r, pl.ds(16 * d, 16)] = acc[d] * inv
        return carry

    lax.fori_loop(0, ROWS_PER_W, row_body, 0)

    pltpu.sync_copy(h_v, h_hbm.at[pl.ds(wid * ROWS_PER_W, ROWS_PER_W)])


_pool = functools.partial(
    pl.kernel,
    mesh=plsc.VectorSubcoreMesh(core_axis_name="c", subcore_axis_name="s"),
    out_type=jax.ShapeDtypeStruct((BATCH, HIDDEN), jnp.float32),
    scratch_types=[
        pltpu.VMEM((CHUNKS_PER_W, CHUNK), jnp.int32),
        pltpu.VMEM((SEQ, HIDDEN), jnp.float32),
        pltpu.VMEM((ROWS_PER_W, HIDDEN), jnp.float32),
        pltpu.SemaphoreType.DMA,
    ],
    compiler_params=pltpu.CompilerParams(use_tc_tiling_on_sc=False),
)(_pool_body)


def _mm_body(h_ref, w_ref, b_ref, o_ref):
    o_ref[...] = lax.dot_general(
        h_ref[...], w_ref[...], (((1,), (1,)), ((), ())),
        preferred_element_type=jnp.float32) + b_ref[...]


def _classify(h, W, b2d):
    return pl.pallas_call(
        _mm_body,
        out_shape=jax.ShapeDtypeStruct((BATCH, LABELS), jnp.float32),
        grid=(8,),
        in_specs=[
            pl.BlockSpec((BATCH // 8, HIDDEN), lambda i: (i, 0)),
            pl.BlockSpec((LABELS, HIDDEN), lambda i: (0, 0)),
            pl.BlockSpec((1, LABELS), lambda i: (0, 0)),
        ],
        out_specs=pl.BlockSpec((BATCH // 8, LABELS), lambda i: (i, 0)),
    )(h, W, b2d)


@jax.jit
def kernel(x, emb, W, b):
    idx = x.astype(jnp.int32).reshape(BATCH * CHUNKS_PER_ROW, CHUNK)
    h = _pool(idx, emb)
    return _classify(h, W, b.reshape(1, LABELS))


# speedup vs baseline: 1.1960x; 1.1960x over previous
import functools

import jax
import jax.numpy as jnp
from jax import lax
from jax.experimental import pallas as pl
from jax.experimental.pallas import tpu as pltpu
from jax.experimental.pallas import tpu_sc as plsc

VOCAB = 1000000
HIDDEN = 64
LABELS = 50
BATCH = 4096
SEQ = 200

NC = 2
NS = 16
NW = NC * NS
ROWS_PER_W = BATCH // NW
CHUNK = 100
CHUNKS_PER_ROW = SEQ // CHUNK
CHUNKS_PER_W = ROWS_PER_W * CHUNKS_PER_ROW
NVEC = HIDDEN // 16


NBUF = 4


def _pool_body(idx_hbm, emb_hbm, h_hbm, idx_v, rows_v, h_v, *sems):
    wid = lax.axis_index("s") * NC + lax.axis_index("c")

    pltpu.sync_copy(idx_hbm.at[pl.ds(wid * CHUNKS_PER_W, CHUNKS_PER_W)], idx_v)

    inv = jnp.float32(1.0 / SEQ)

    def fire(r, b):
        c0 = r * CHUNKS_PER_ROW
        pltpu.async_copy(
            emb_hbm.at[idx_v.at[c0]], rows_v.at[b].at[pl.ds(0, CHUNK)],
            sems[b])
        pltpu.async_copy(
            emb_hbm.at[idx_v.at[c0 + 1]], rows_v.at[b].at[pl.ds(CHUNK, CHUNK)],
            sems[b])

    def drain(b):
        pltpu.make_async_copy(
            emb_hbm.at[idx_v.at[0]], rows_v.at[b].at[pl.ds(0, CHUNK)],
            sems[b]).wait()
        pltpu.make_async_copy(
            emb_hbm.at[idx_v.at[0]], rows_v.at[b].at[pl.ds(CHUNK, CHUNK)],
            sems[b]).wait()

    for b in range(NBUF):
        fire(b, b)

    @pl.loop(0, ROWS_PER_W, step=NBUF)
    def _outer(r0):
        for b in range(NBUF):
            r = r0 + b
            drain(b)

            def acc_body(j, acc):
                return tuple(
                    acc[d] + rows_v[b, j, pl.ds(16 * d, 16)]
                    for d in range(NVEC))

            acc = lax.fori_loop(
                0, SEQ, acc_body,
                tuple(jnp.zeros((16,), jnp.float32) for _ in range(NVEC)),
                unroll=8)
            for d in range(NVEC):
                h_v[r, pl.ds(16 * d, 16)] = acc[d] * inv

            nxt = r + NBUF

            @pl.when(nxt < ROWS_PER_W)
            def _():
                fire(nxt, b)

    pltpu.sync_copy(h_v, h_hbm.at[pl.ds(wid * ROWS_PER_W, ROWS_PER_W)])


_pool = functools.partial(
    pl.kernel,
    mesh=plsc.VectorSubcoreMesh(core_axis_name="c", subcore_axis_name="s"),
    out_type=jax.ShapeDtypeStruct((BATCH, HIDDEN), jnp.float32),
    scratch_types=[
        pltpu.VMEM((CHUNKS_PER_W, CHUNK), jnp.int32),
        pltpu.VMEM((NBUF, SEQ, HIDDEN), jnp.float32),
        pltpu.VMEM((ROWS_PER_W, HIDDEN), jnp.float32),
    ] + [pltpu.SemaphoreType.DMA] * NBUF,
    compiler_params=pltpu.CompilerParams(use_tc_tiling_on_sc=False),
)(_pool_body)


def _mm_body(h_ref, w_ref, b_ref, o_ref):
    o_ref[...] = lax.dot_general(
        h_ref[...], w_ref[...], (((1,), (1,)), ((), ())),
        preferred_element_type=jnp.float32) + b_ref[...]


def _classify(h, W, b2d):
    return pl.pallas_call(
        _mm_body,
        out_shape=jax.ShapeDtypeStruct((BATCH, LABELS), jnp.float32),
        grid=(8,),
        in_specs=[
            pl.BlockSpec((BATCH // 8, HIDDEN), lambda i: (i, 0)),
            pl.BlockSpec((LABELS, HIDDEN), lambda i: (0, 0)),
            pl.BlockSpec((1, LABELS), lambda i: (0, 0)),
        ],
        out_specs=pl.BlockSpec((BATCH // 8, LABELS), lambda i: (i, 0)),
    )(h, W, b2d)


@jax.jit
def kernel(x, emb, W, b):
    idx = x.astype(jnp.int32).reshape(BATCH * CHUNKS_PER_ROW, CHUNK)
    h = _pool(idx, emb)
    return _classify(h, W, b.reshape(1, LABELS))
